# baseline (speedup 1.0000x reference)
import functools

import jax
import jax.numpy as jnp
from jax import lax
from jax.experimental import pallas as pl
from jax.experimental.pallas import tpu as pltpu
from jax.experimental.pallas import tpu_sc as plsc

B = 4096
L = 20
D = 64
NT = 4
NC = 2
NS = 16
NW = NC * NS
BW = B // NW
BC = 32
CH = BW // BC
NU = NT * CH
UR = BC * L
PAD = 0


def _fix_pad_rows(idx_v, rows, t, u0, j):
    iv = idx_v[t, pl.ds(u0 + j * 16, 16)]
    anyp = jnp.any(iv == PAD)

    @pl.when(anyp)
    def _():
        m_f = jnp.where(iv == PAD, jnp.float32(0), jnp.float32(1))
        dnums = lax.GatherDimensionNumbers(
            offset_dims=(), collapsed_slice_dims=(0,), start_index_map=(0,))
        for r in range(16):
            bc = lax.gather(
                m_f, jnp.full((16, 1), r, jnp.int32),
                dimension_numbers=dnums, slice_sizes=(1,),
                mode=lax.GatherScatterMode.PROMISE_IN_BOUNDS)
            row = j * 16 + r
            for cc in range(D // 16):
                sl = pl.ds(cc * 16, 16)
                rows[row, sl] = rows[row, sl] * bc


def _emb_body(src_hbm, node_hbm, tok_hbm, act_hbm,
              w_src, w_node, w_tok, w_act,
              out_hbm,
              idx_v, rows_v, gsem0, gsem1, wsem0, wsem1):
    c = lax.axis_index("c")
    s = lax.axis_index("s")
    wid = s * NC + c
    b0 = wid * BW

    ids = (src_hbm, node_hbm, tok_hbm, act_hbm)
    for t in range(NT):
        pltpu.sync_copy(ids[t].at[pl.ds(b0 * L, BW * L)], idx_v.at[t])

    tables = (w_src, w_node, w_tok, w_act)
    gsems = (gsem0, gsem1)
    wsems = (wsem0, wsem1)

    def ga(u, p):
        t, ch = u // CH, u % CH
        return pltpu.make_async_copy(
            tables[t].at[idx_v.at[t, pl.ds(ch * UR, UR)]],
            rows_v.at[p], gsems[p])

    def wr(u, p):
        t, ch = u // CH, u % CH
        return pltpu.make_async_copy(
            rows_v.at[p],
            out_hbm.at[pl.ds((b0 + ch * BC) * L, UR), pl.ds(t * D, D)],
            wsems[p])

    def fix(u, p):
        t, ch = u // CH, u % CH

        def grp(j, _, t=t, ch=ch, p=p):
            _fix_pad_rows(idx_v, rows_v.at[p], t, ch * UR, j)
            return 0
        lax.fori_loop(0, UR // 16, grp, 0, unroll=False)

    ga(0, 0).start()
    for u in range(NU):
        p = u % 2
        q = 1 - p
        if u + 1 < NU:
            if u >= 1:
                wr(u - 1, q).wait()
            ga(u + 1, q).start()
        ga(u, p).wait()
        fix(u, p)
        wr(u, p).start()
    wr(NU - 2, NU % 2).wait()
    wr(NU - 1, 1 - NU % 2).wait()


@functools.partial(
    pl.kernel,
    mesh=plsc.VectorSubcoreMesh(core_axis_name="c", subcore_axis_name="s"),
    out_type=jax.ShapeDtypeStruct((B * L, NT * D), jnp.float32),
    scratch_types=[
        pltpu.VMEM((NT, BW * L), jnp.int32),
        pltpu.VMEM((2, UR, D), jnp.float32),
        pltpu.SemaphoreType.DMA,
        pltpu.SemaphoreType.DMA,
        pltpu.SemaphoreType.DMA,
        pltpu.SemaphoreType.DMA,
    ],
    compiler_params=pltpu.CompilerParams(use_tc_tiling_on_sc=False,
                                         needs_layout_passes=False),
)
def _emb_lookup(*refs):
    _emb_body(*refs)


def kernel(src_ids, tgt_node_ids, tgt_token_ids, tgt_action_ids,
           W_src, W_node, W_tok, W_act):
    out = _emb_lookup(src_ids.reshape(-1), tgt_node_ids.reshape(-1),
                      tgt_token_ids.reshape(-1), tgt_action_ids.reshape(-1),
                      W_src, W_node, W_tok, W_act)
    return out.reshape(B, L, NT * D)


# speedup vs baseline: 1.1718x; 1.1718x over previous
import functools

import jax
import jax.numpy as jnp
from jax import lax
from jax.experimental import pallas as pl
from jax.experimental.pallas import tpu as pltpu
from jax.experimental.pallas import tpu_sc as plsc

B = 4096
L = 20
D = 64
NT = 4
NC = 2
NS = 16
NW = NC * NS
BW = B // NW
PAD = 0


def _fix_pad_rows(idx_v, rows, t, l, j):
    iv = idx_v[t, l, pl.ds(j * 16, 16)]
    anyp = jnp.any(iv == PAD)

    @pl.when(anyp)
    def _():
        m_f = jnp.where(iv == PAD, jnp.float32(0), jnp.float32(1))
        dnums = lax.GatherDimensionNumbers(
            offset_dims=(), collapsed_slice_dims=(0,), start_index_map=(0,))
        for r in range(16):
            bc = lax.gather(
                m_f, jnp.full((16, 1), r, jnp.int32),
                dimension_numbers=dnums, slice_sizes=(1,),
                mode=lax.GatherScatterMode.PROMISE_IN_BOUNDS)
            row = j * 16 + r
            for cc in range(D // 16):
                sl = pl.ds(cc * 16, 16)
                rows[row, sl] = rows[row, sl] * bc


def _emb_body(src_hbm, node_hbm, tok_hbm, act_hbm,
              w_src, w_node, w_tok, w_act,
              out_hbm,
              idx_v, rows_v, gsem0, gsem1, gsem2, gsem3,
              wsem0, wsem1, wsem2, wsem3):
    c = lax.axis_index("c")
    s = lax.axis_index("s")
    wid = s * NC + c
    b0 = wid * BW

    ids = (src_hbm, node_hbm, tok_hbm, act_hbm)
    for t in range(NT):
        pltpu.sync_copy(ids[t].at[:, pl.ds(b0, BW)], idx_v.at[t])

    tables = (w_src, w_node, w_tok, w_act)
    gsems = (gsem0, gsem1, gsem2, gsem3)
    wsems = (wsem0, wsem1, wsem2, wsem3)

    def ga(t, l, p):
        return pltpu.make_async_copy(
            tables[t].at[idx_v.at[t, l, :]], rows_v.at[p], gsems[p])

    def wr(t, l, p):
        return pltpu.make_async_copy(
            rows_v.at[p],
            out_hbm.at[l, pl.ds(b0, BW), pl.ds(t * D, D)], wsems[p])

    for t in range(NT):
        ga(t, 0, 0).start()

        def body(k, _, t=t):
            for i in range(4):
                l = 4 * k + i
                p = i
                q = (i + 1) % 4
                if i < 3:
                    @pl.when(k >= 1)
                    def _(t=t, l=l, q=q):
                        wr(t, l - 3, q).wait()
                    ga(t, l + 1, q).start()
                else:
                    wr(t, l - 3, q).wait()

                    @pl.when(k < (L // 4) - 1)
                    def _(t=t, l=l, q=q):
                        ga(t, l + 1, q).start()
                ga(t, l, p).wait()

                def grp(j, _, t=t, l=l, p=p):
                    _fix_pad_rows(idx_v, rows_v.at[p], t, l, j)
                    return 0
                lax.fori_loop(0, BW // 16, grp, 0, unroll=False)
                wr(t, l, p).start()
            return 0

        lax.fori_loop(0, L // 4, body, 0, unroll=False)
        for i in range(1, 4):
            wr(t, L - 4 + i, i).wait()


@functools.partial(
    pl.kernel,
    mesh=plsc.VectorSubcoreMesh(core_axis_name="c", subcore_axis_name="s"),
    out_type=jax.ShapeDtypeStruct((L, B, NT * D), jnp.float32),
    scratch_types=[
        pltpu.VMEM((NT, L, BW), jnp.int32),
        pltpu.VMEM((4, BW, D), jnp.float32),
        pltpu.SemaphoreType.DMA,
        pltpu.SemaphoreType.DMA,
        pltpu.SemaphoreType.DMA,
        pltpu.SemaphoreType.DMA,
        pltpu.SemaphoreType.DMA,
        pltpu.SemaphoreType.DMA,
        pltpu.SemaphoreType.DMA,
        pltpu.SemaphoreType.DMA,
    ],
    compiler_params=pltpu.CompilerParams(use_tc_tiling_on_sc=False,
                                         needs_layout_passes=False),
)
def _emb_lookup(*refs):
    _emb_body(*refs)


def kernel(src_ids, tgt_node_ids, tgt_token_ids, tgt_action_ids,
           W_src, W_node, W_tok, W_act):
    out = _emb_lookup(src_ids.T, tgt_node_ids.T, tgt_token_ids.T,
                      tgt_action_ids.T, W_src, W_node, W_tok, W_act)
    return jnp.transpose(out, (1, 0, 2))
